# Initial kernel scaffold; baseline (speedup 1.0000x reference)
import functools

import jax
import jax.numpy as jnp
from jax import lax
from jax.experimental import pallas as pl
from jax.experimental.pallas import tpu as pltpu
from jax.experimental.pallas import tpu_sc as plsc

N = 10000
E = 320000
IN = 128
OUT = 64
H1 = 42
H2 = 21

NC = 2
NS = 16
NW = NC * NS
L = 16

NP = 10240
EPW = E // NW
C = 80
NCHUNK = EPW // C
GPC = C // L
ROWS_PT = NP // NS


def _sc_edge_body(xl_hbm, xr_hbm, att_hbm, ei_hbm,
                  den_out, oun_out,
                  src_v, dst_v, xl_rows, xr_rows, sc_buf, att_v, den_v,
                  red_v, out_sh, den_sh, sem1, sem2):
    c = lax.axis_index("c")
    s = lax.axis_index("s")
    wid = s * NC + c

    zero16 = jnp.zeros((L,), jnp.float32)

    @pl.loop(0, NP // L)
    def _(j):
        den_v[pl.ds(j * L, L)] = zero16

    @pl.loop(0, C * OUT // L)
    def _(j):
        r = j // (OUT // L)
        k = j % (OUT // L)
        sc_buf[r, pl.ds(k * L, L)] = zero16

    for j in range(ROWS_PT // C):
        pltpu.sync_copy(sc_buf, out_sh.at[pl.ds(s * ROWS_PT + j * C, C)])

    pltpu.sync_copy(att_hbm, att_v)
    plsc.subcore_barrier()

    @pl.loop(0, NCHUNK)
    def _(i):
        base = wid * EPW + i * C
        pltpu.sync_copy(ei_hbm.at[0, pl.ds(base, C)], src_v)
        pltpu.sync_copy(ei_hbm.at[1, pl.ds(base, C)], dst_v)
        cp1 = pltpu.async_copy(xl_hbm.at[src_v], xl_rows, sem1)
        cp2 = pltpu.async_copy(xr_hbm.at[dst_v], xr_rows, sem2)
        cp1.wait()
        cp2.wait()

        eids = [jax.lax.iota(jnp.int32, L) + g * L for g in range(GPC)]
        accs = [jnp.zeros((L,), jnp.float32) for _ in range(GPC)]
        for d in range(OUT):
            dsplat = jnp.full((L,), d, jnp.int32)
            attd = plsc.load_gather(att_v, [dsplat])
            for g in range(GPC):
                a = plsc.load_gather(xl_rows, [eids[g], dsplat])
                b = plsc.load_gather(xr_rows, [eids[g], dsplat])
                z = a + b
                lz = jnp.maximum(z, 0.2 * z)
                accs[g] = accs[g] + attd * lz

        for g in range(GPC):
            ex = jnp.exp(accs[g])
            dst_g = dst_v[pl.ds(g * L, L)]
            plsc.addupdate_scatter(den_v, [dst_g], ex)
            for e in range(L):
                row = g * L + e
                exe = jnp.broadcast_to(lax.slice(ex, (e,), (e + 1,)), (L,))
                for k in range(OUT // L):
                    sc_buf[row, pl.ds(k * L, L)] = (
                        exe * xl_rows[row, pl.ds(k * L, L)])

        pltpu.sync_copy(sc_buf, out_sh.at[dst_v], add=True)

    pltpu.sync_copy(den_v, den_sh.at[s])
    plsc.subcore_barrier()
    col0 = s * ROWS_PT
    pltpu.sync_copy(den_sh.at[:, pl.ds(col0, ROWS_PT)], red_v)

    @pl.loop(0, ROWS_PT // L)
    def _(j):
        acc = red_v[0, pl.ds(j * L, L)]
        for t in range(1, NS):
            acc = acc + red_v[t, pl.ds(j * L, L)]
        den_v[pl.ds(j * L, L)] = acc

    pltpu.sync_copy(den_v.at[pl.ds(0, ROWS_PT)],
                    den_out.at[c, pl.ds(col0, ROWS_PT)])
    pltpu.sync_copy(out_sh.at[pl.ds(col0, ROWS_PT)],
                    oun_out.at[c, pl.ds(col0, ROWS_PT)])


def _sc_edge(xl, xr, att, ei):
    mesh = plsc.VectorSubcoreMesh(core_axis_name="c", subcore_axis_name="s")
    f = pl.kernel(
        _sc_edge_body,
        out_type=[
            jax.ShapeDtypeStruct((NC, NP), jnp.float32),
            jax.ShapeDtypeStruct((NC, NP, OUT), jnp.float32),
        ],
        mesh=mesh,
        scratch_types=[
            pltpu.VMEM((C,), jnp.int32),
            pltpu.VMEM((C,), jnp.int32),
            pltpu.VMEM((C, OUT), jnp.float32),
            pltpu.VMEM((C, OUT), jnp.float32),
            pltpu.VMEM((C, OUT), jnp.float32),
            pltpu.VMEM((OUT,), jnp.float32),
            pltpu.VMEM((NP,), jnp.float32),
            pltpu.VMEM((NS, ROWS_PT), jnp.float32),
            pltpu.VMEM_SHARED((NP, OUT), jnp.float32),
            pltpu.VMEM_SHARED((NS, NP), jnp.float32),
            pltpu.SemaphoreType.DMA,
            pltpu.SemaphoreType.DMA,
        ],
    )
    return f(xl, xr, att, ei)


def _bn_cols(x, g, b, eps=1e-5):
    m = jnp.mean(x, axis=0, keepdims=True)
    v = jnp.mean((x - m) ** 2, axis=0, keepdims=True)
    return g * (x - m) * jax.lax.rsqrt(v + eps) + b


def _tc1_body(x_ref, gin_ref, bin_ref, w_ref, b_ref, y_ref, q_ref):
    xn = _bn_cols(x_ref[...], gin_ref[...], bin_ref[...])
    y = jnp.dot(xn, w_ref[...], preferred_element_type=jnp.float32) + b_ref[...]
    y_ref[...] = y[:, : 6 * OUT]
    q_ref[...] = jnp.tanh(y[:, 6 * OUT:])


def _tc1(x, gin, bin_, w_all, b_all):
    return pl.pallas_call(
        _tc1_body,
        out_shape=(
            jax.ShapeDtypeStruct((N, 6 * OUT), jnp.float32),
            jax.ShapeDtypeStruct((N, OUT), jnp.float32),
        ),
    )(x, gin.reshape(1, IN), bin_.reshape(1, IN), w_all, b_all.reshape(1, -1))


def _tc2_body(dp_ref, op_ref, ds_ref, os_ref, dv_ref, ov_ref,
              biases_ref, gb_ref, q_ref,
              wk_ref, bk_ref, wv_ref, bv_ref,
              wc1_ref, bc1_ref, g1_ref, bn1_ref,
              wc2_ref, bc2_ref, g2_ref, bn2_ref,
              wc3_ref, bc3_ref, out_ref):
    q = q_ref[...]
    ws = []
    vals = []
    for idx, (d_ref, o_ref) in enumerate(
            ((dp_ref, op_ref), (ds_ref, os_ref), (dv_ref, ov_ref))):
        den = d_ref[0, :N] + d_ref[1, :N] + 1e-16
        oun = o_ref[0, :N, :] + o_ref[1, :N, :]
        o = oun / den[:, None] + biases_ref[idx, 0:1, :]
        e = jnp.tanh(_bn_cols(o, gb_ref[idx, 0:1, :], gb_ref[idx, 1:2, :]))
        keys = jnp.tanh(
            jnp.dot(e, wk_ref[...], preferred_element_type=jnp.float32)
            + bk_ref[...])
        ws.append(jnp.sum(keys * q, axis=1, keepdims=True))
        vals.append(jnp.tanh(
            jnp.dot(e, wv_ref[...], preferred_element_type=jnp.float32)
            + bv_ref[...]))
    m = jnp.maximum(jnp.maximum(ws[0], ws[1]), ws[2])
    es = [jnp.exp(w - m) for w in ws]
    tot = es[0] + es[1] + es[2]
    r = (es[0] * vals[0] + es[1] * vals[1] + es[2] * vals[2]) / tot
    h = jnp.tanh(_bn_cols(
        jnp.dot(r, wc1_ref[...], preferred_element_type=jnp.float32)
        + bc1_ref[...], g1_ref[...], bn1_ref[...]))
    h = jnp.tanh(_bn_cols(
        jnp.dot(h, wc2_ref[...], preferred_element_type=jnp.float32)
        + bc2_ref[...], g2_ref[...], bn2_ref[...]))
    out_ref[...] = (
        jnp.dot(h, wc3_ref[...], preferred_element_type=jnp.float32)
        + bc3_ref[...])


def _tc2(parts, biases, gbs, q, p):
    args = []
    for (den, oun) in parts:
        args += [den, oun]
    return pl.pallas_call(
        _tc2_body,
        out_shape=jax.ShapeDtypeStruct((N, 2), jnp.float32),
    )(*args, biases, gbs, q,
      p['Wk'], p['bk'].reshape(1, OUT), p['Wv'], p['bv'].reshape(1, OUT),
      p['Wc1'], p['bc1'].reshape(1, H1), p['gc1'].reshape(1, H1),
      p['bnc1'].reshape(1, H1),
      p['Wc2'], p['bc2'].reshape(1, H2), p['gc2'].reshape(1, H2),
      p['bnc2'].reshape(1, H2),
      p['Wc3'], p['bc3'].reshape(1, 2))


def kernel(x, edge_index_p, edge_index_s, edge_index_v, params):
    p = params
    w_all = jnp.concatenate(
        [p['p_Wl'], p['p_Wr'], p['s_Wl'], p['s_Wr'],
         p['v_Wl'], p['v_Wr'], p['Wq']], axis=1)
    b_all = jnp.concatenate(
        [p['p_bl'], p['p_br'], p['s_bl'], p['s_br'],
         p['v_bl'], p['v_br'], p['bq']], axis=0)
    y, q = _tc1(x, p['g_in'], p['b_in'], w_all, b_all)

    parts = []
    for i, (pref, ei) in enumerate(
            (('p', edge_index_p), ('s', edge_index_s), ('v', edge_index_v))):
        xl = y[:, 2 * i * OUT:(2 * i + 1) * OUT]
        xr = y[:, (2 * i + 1) * OUT:(2 * i + 2) * OUT]
        den, oun = _sc_edge(xl, xr, p[pref + '_att'], ei)
        parts.append((den, oun))

    biases = jnp.stack(
        [p['p_bias'].reshape(1, OUT), p['s_bias'].reshape(1, OUT),
         p['v_bias'].reshape(1, OUT)], axis=0)
    gbs = jnp.stack(
        [jnp.stack([p['p_g'], p['p_b']]), jnp.stack([p['s_g'], p['s_b']]),
         jnp.stack([p['v_g'], p['v_b']])], axis=0)
    return _tc2(parts, biases, gbs, q, params)


# speedup vs baseline: 4.9739x; 4.9739x over previous
import functools

import jax
import jax.numpy as jnp
from jax import lax
from jax.experimental import pallas as pl
from jax.experimental.pallas import tpu as pltpu
from jax.experimental.pallas import tpu_sc as plsc

N = 10000
E = 320000
IN = 128
OUT = 64
H1 = 42
H2 = 21

NC = 2
NS = 16
NW = NC * NS
L = 16

NP = 10240
EPW = E // NW
C = 80
NCHUNK = EPW // C
GPC = C // L
ROWS_PT = NP // NS


def _sc_edge_body(xl_hbm, xr_hbm, att_hbm, src_hbm, dst_hbm,
                  den_out, oun_out,
                  src_v, dst_v, xl_rows, xr_rows, sc_buf, att_v, ex_v, den_v,
                  red_v, out_sh, den_sh, sem1, sem2):
    c = lax.axis_index("c")
    s = lax.axis_index("s")
    wid = s * NC + c

    zero16 = jnp.zeros((L,), jnp.float32)

    @pl.loop(0, NP // L)
    def _(j):
        den_v[pl.ds(j * L, L)] = zero16

    @pl.loop(0, C * OUT // L)
    def _(j):
        r = j // (OUT // L)
        k = j % (OUT // L)
        sc_buf[r, pl.ds(k * L, L)] = zero16

    for j in range(ROWS_PT // C):
        pltpu.sync_copy(sc_buf, out_sh.at[pl.ds(s * ROWS_PT + j * C, C)])

    pltpu.sync_copy(att_hbm, att_v.at[pl.ds(L, OUT)])
    plsc.subcore_barrier()

    @pl.loop(0, NCHUNK)
    def _(i):
        base = wid * EPW + i * C
        pltpu.sync_copy(src_hbm.at[pl.ds(base, C)], src_v)
        pltpu.sync_copy(dst_hbm.at[pl.ds(base, C)], dst_v)
        cp1 = pltpu.async_copy(xl_hbm.at[src_v], xl_rows, sem1)
        cp2 = pltpu.async_copy(xr_hbm.at[dst_v], xr_rows, sem2)
        cp1.wait()
        cp2.wait()

        eids = [jax.lax.iota(jnp.int32, L) + g * L for g in range(GPC)]
        accs = [jnp.zeros((L,), jnp.float32) for _ in range(GPC)]
        for d in range(OUT):
            dsplat = jnp.full((L,), d, jnp.int32)
            attd = plsc.load_gather(att_v, [jnp.full((L,), L + d, jnp.int32)])
            for g in range(GPC):
                a = plsc.load_gather(xl_rows, [eids[g], dsplat])
                b = plsc.load_gather(xr_rows, [eids[g], dsplat])
                z = a + b
                lz = jnp.maximum(z, 0.2 * z)
                accs[g] = accs[g] + attd * lz

        for g in range(GPC):
            ex = jnp.exp(accs[g])
            dst_g = dst_v[pl.ds(g * L, L)]
            plsc.addupdate_scatter(den_v, [dst_g], ex)
            ex_v[pl.ds(L, L)] = ex
            for e in range(L):
                row = g * L + e
                exe = plsc.load_gather(ex_v, [jnp.full((L,), L + e, jnp.int32)])
                for k in range(OUT // L):
                    sc_buf[row, pl.ds(k * L, L)] = (
                        exe * xl_rows[row, pl.ds(k * L, L)])

        pltpu.sync_copy(sc_buf, out_sh.at[dst_v], add=True)

    pltpu.sync_copy(den_v, den_sh.at[s])
    plsc.subcore_barrier()
    col0 = s * ROWS_PT
    pltpu.sync_copy(den_sh.at[:, pl.ds(col0, ROWS_PT)], red_v)

    @pl.loop(0, ROWS_PT // L)
    def _(j):
        acc = red_v[0, pl.ds(j * L, L)]
        for t in range(1, NS):
            acc = acc + red_v[t, pl.ds(j * L, L)]
        den_v[pl.ds(j * L, L)] = acc

    pltpu.sync_copy(den_v.at[pl.ds(0, ROWS_PT)],
                    den_out.at[c, pl.ds(col0, ROWS_PT)])
    pltpu.sync_copy(out_sh.at[pl.ds(col0, ROWS_PT)],
                    oun_out.at[c, pl.ds(col0, ROWS_PT)])


def _sc_edge(xl, xr, att, src, dst):
    mesh = plsc.VectorSubcoreMesh(core_axis_name="c", subcore_axis_name="s")
    f = pl.kernel(
        _sc_edge_body,
        out_type=[
            jax.ShapeDtypeStruct((NC, NP), jnp.float32),
            jax.ShapeDtypeStruct((NC, NP, OUT), jnp.float32),
        ],
        mesh=mesh,
        compiler_params=pltpu.CompilerParams(
            needs_layout_passes=False, use_tc_tiling_on_sc=False),
        scratch_types=[
            pltpu.VMEM((C,), jnp.int32),
            pltpu.VMEM((C,), jnp.int32),
            pltpu.VMEM((C, OUT), jnp.float32),
            pltpu.VMEM((C, OUT), jnp.float32),
            pltpu.VMEM((C, OUT), jnp.float32),
            pltpu.VMEM((OUT + L,), jnp.float32),
            pltpu.VMEM((2 * L,), jnp.float32),
            pltpu.VMEM((NP,), jnp.float32),
            pltpu.VMEM((NS, ROWS_PT), jnp.float32),
            pltpu.VMEM_SHARED((NP, OUT), jnp.float32),
            pltpu.VMEM_SHARED((NS, NP), jnp.float32),
            pltpu.SemaphoreType.DMA,
            pltpu.SemaphoreType.DMA,
        ],
    )
    return f(xl, xr, att, src, dst)


def _bn_cols(x, g, b, eps=1e-5):
    m = jnp.mean(x, axis=0, keepdims=True)
    v = jnp.mean((x - m) ** 2, axis=0, keepdims=True)
    return g * (x - m) * jax.lax.rsqrt(v + eps) + b


def _tc1_body(x_ref, gin_ref, bin_ref, w_ref, b_ref, y_ref, q_ref):
    xn = _bn_cols(x_ref[...], gin_ref[...], bin_ref[...])
    y = jnp.dot(xn, w_ref[...], preferred_element_type=jnp.float32) + b_ref[...]
    y_ref[...] = y[:, : 6 * OUT]
    q_ref[...] = jnp.tanh(y[:, 6 * OUT:])


def _tc1(x, gin, bin_, w_all, b_all):
    return pl.pallas_call(
        _tc1_body,
        out_shape=(
            jax.ShapeDtypeStruct((N, 6 * OUT), jnp.float32),
            jax.ShapeDtypeStruct((N, OUT), jnp.float32),
        ),
    )(x, gin.reshape(1, IN), bin_.reshape(1, IN), w_all, b_all.reshape(1, -1))


def _tc2a_body(d_ref, o_ref, bias_ref, g_ref, b_ref, q_ref,
               wk_ref, bk_ref, wv_ref, bv_ref, s_ref, vals_ref):
    den = d_ref[0, :N] + d_ref[1, :N] + 1e-16
    oun = o_ref[0, :N, :] + o_ref[1, :N, :]
    o = oun / den[:, None] + bias_ref[...]
    e = jnp.tanh(_bn_cols(o, g_ref[...], b_ref[...]))
    keys = jnp.tanh(
        jnp.dot(e, wk_ref[...], preferred_element_type=jnp.float32)
        + bk_ref[...])
    s_ref[...] = jnp.sum(keys * q_ref[...], axis=1, keepdims=True)
    vals_ref[...] = jnp.tanh(
        jnp.dot(e, wv_ref[...], preferred_element_type=jnp.float32)
        + bv_ref[...])


def _tc2a(den, oun, bias, g, b, q, p):
    return pl.pallas_call(
        _tc2a_body,
        out_shape=(
            jax.ShapeDtypeStruct((N, 1), jnp.float32),
            jax.ShapeDtypeStruct((N, OUT), jnp.float32),
        ),
    )(den, oun, bias.reshape(1, OUT), g.reshape(1, OUT), b.reshape(1, OUT),
      q, p['Wk'], p['bk'].reshape(1, OUT), p['Wv'], p['bv'].reshape(1, OUT))


def _tc2b_body(sp_ref, vp_ref, ss_ref, vs_ref, sv_ref, vv_ref,
               wc1_ref, bc1_ref, g1_ref, bn1_ref,
               wc2_ref, bc2_ref, g2_ref, bn2_ref,
               wc3_ref, bc3_ref, out_ref):
    ws = [sp_ref[...], ss_ref[...], sv_ref[...]]
    vals = [vp_ref[...], vs_ref[...], vv_ref[...]]
    m = jnp.maximum(jnp.maximum(ws[0], ws[1]), ws[2])
    es = [jnp.exp(w - m) for w in ws]
    tot = es[0] + es[1] + es[2]
    r = (es[0] * vals[0] + es[1] * vals[1] + es[2] * vals[2]) / tot
    h = jnp.tanh(_bn_cols(
        jnp.dot(r, wc1_ref[...], preferred_element_type=jnp.float32)
        + bc1_ref[...], g1_ref[...], bn1_ref[...]))
    h = jnp.tanh(_bn_cols(
        jnp.dot(h, wc2_ref[...], preferred_element_type=jnp.float32)
        + bc2_ref[...], g2_ref[...], bn2_ref[...]))
    out_ref[...] = (
        jnp.dot(h, wc3_ref[...], preferred_element_type=jnp.float32)
        + bc3_ref[...])


def _tc2b(svs, p):
    args = []
    for (s, v) in svs:
        args += [s, v]
    return pl.pallas_call(
        _tc2b_body,
        out_shape=jax.ShapeDtypeStruct((N, 2), jnp.float32),
    )(*args,
      p['Wc1'], p['bc1'].reshape(1, H1), p['gc1'].reshape(1, H1),
      p['bnc1'].reshape(1, H1),
      p['Wc2'], p['bc2'].reshape(1, H2), p['gc2'].reshape(1, H2),
      p['bnc2'].reshape(1, H2),
      p['Wc3'], p['bc3'].reshape(1, 2))


def kernel(x, edge_index_p, edge_index_s, edge_index_v, params):
    p = params
    w_all = jnp.concatenate(
        [p['p_Wl'], p['p_Wr'], p['s_Wl'], p['s_Wr'],
         p['v_Wl'], p['v_Wr'], p['Wq']], axis=1)
    b_all = jnp.concatenate(
        [p['p_bl'], p['p_br'], p['s_bl'], p['s_br'],
         p['v_bl'], p['v_br'], p['bq']], axis=0)
    y, q = _tc1(x, p['g_in'], p['b_in'], w_all, b_all)

    svs = []
    for i, (pref, ei) in enumerate(
            (('p', edge_index_p), ('s', edge_index_s), ('v', edge_index_v))):
        xl = y[:, 2 * i * OUT:(2 * i + 1) * OUT]
        xr = y[:, (2 * i + 1) * OUT:(2 * i + 2) * OUT]
        den, oun = _sc_edge(xl, xr, p[pref + '_att'], ei[0], ei[1])
        svs.append(_tc2a(den, oun, p[pref + '_bias'], p[pref + '_g'],
                         p[pref + '_b'], q, p))

    return _tc2b(svs, params)
